# baseline (speedup 1.0000x reference)
import functools
import math

import jax
import jax.numpy as jnp
from jax import lax
from jax.experimental import pallas as pl
from jax.experimental.pallas import tpu as pltpu
from jax.experimental.pallas import tpu_sc as plsc

_NC, _NS = 2, 16
_NW = _NC * _NS


def _cdiv(a, b):
  return (a + b - 1) // b


def _mm(x, w, *, bias=None, act=None, mask=None, mb=None):
  M, K = x.shape
  N = w.shape[1]
  if mb is None or mb >= M:
    mb = M
  grid = (_cdiv(M, mb),)
  in_specs = [
      pl.BlockSpec((mb, K), lambda i: (i, 0)),
      pl.BlockSpec((K, N), lambda i: (0, 0)),
  ]
  args = [x, w]
  if bias is not None:
    in_specs.append(pl.BlockSpec((1, N), lambda i: (0, 0)))
    args.append(bias.reshape(1, N))
  if mask is not None:
    in_specs.append(pl.BlockSpec((mb, 1), lambda i: (i, 0)))
    args.append(mask)

  def kern(*refs):
    x_ref, w_ref = refs[0], refs[1]
    o_ref = refs[-1]
    acc = jnp.dot(x_ref[...], w_ref[...], preferred_element_type=jnp.float32)
    p = 2
    if bias is not None:
      acc = acc + refs[p][...]
      p += 1
    if act == 'elu':
      acc = jnp.where(acc > 0, acc, jnp.exp(jnp.minimum(acc, 0.0)) - 1.0)
    if mask is not None:
      acc = acc * refs[p][...]
      p += 1
    o_ref[...] = acc

  return pl.pallas_call(
      kern,
      grid=grid,
      in_specs=in_specs,
      out_specs=pl.BlockSpec((mb, N), lambda i: (i, 0)),
      out_shape=jax.ShapeDtypeStruct((M, N), jnp.float32),
  )(*args)


def _sc_gather(table, gidx, F, C, rpw):
  Gp = gidx.shape[0]
  assert Gp == _NW * rpw * C
  mesh = plsc.VectorSubcoreMesh(
      core_axis_name="c", subcore_axis_name="s",
      num_cores=_NC, num_subcores=_NS)

  @functools.partial(
      pl.kernel,
      out_type=jax.ShapeDtypeStruct((Gp, F), jnp.float32),
      mesh=mesh,
      scratch_types=[
          pltpu.VMEM((rpw * C,), jnp.int32),
          pltpu.VMEM((C, F), jnp.float32),
          pltpu.VMEM((C, F), jnp.float32),
          pltpu.SemaphoreType.DMA,
          pltpu.SemaphoreType.DMA,
          pltpu.SemaphoreType.DMA,
          pltpu.SemaphoreType.DMA,
      ],
  )
  def gk(table_hbm, gidx_hbm, out_hbm, idx_v, buf0, buf1, is0, is1, os0, os1):
    wid = lax.axis_index("s") * _NC + lax.axis_index("c")
    base = wid * (rpw * C)
    pltpu.sync_copy(gidx_hbm.at[pl.ds(base, rpw * C)], idx_v)
    bufs = (buf0, buf1)
    isems = (is0, is1)
    osems = (os0, os1)

    def fire_in(j):
      return pltpu.async_copy(
          table_hbm.at[idx_v.at[pl.ds(j * C, C)]], bufs[j % 2], isems[j % 2])

    def fire_out(j):
      return pltpu.async_copy(
          bufs[j % 2], out_hbm.at[pl.ds(base + j * C, C)], osems[j % 2])

    ins = [None] * rpw
    outs = [None] * rpw
    ins[0] = fire_in(0)
    for j in range(rpw):
      if j + 1 < rpw:
        if j - 1 >= 0:
          outs[j - 1].wait()
        ins[j + 1] = fire_in(j + 1)
      ins[j].wait()
      outs[j] = fire_out(j)
    if rpw >= 2:
      outs[rpw - 2].wait()
    outs[rpw - 1].wait()

  return gk(table, gidx)


_LEVEL_TUNE = [
    (96, None, None),
    (120, None, 640),
    (400, 1112, 1280),
]


def kernel(x, W_fc, b_fc, U0, U1, U2, S0, S1, S2, Wc0, bc0, Wc1, bc1, Wc2, bc2):
  B = x.shape[0]
  h = _mm(x, W_fc, bias=b_fc)
  M_in = U2.shape[1]
  F_in = h.shape[1] // M_in
  hcol = h.reshape(B, M_in, F_in).transpose(1, 0, 2).reshape(M_in, B * F_in)

  specs = [
      (U2, S2, Wc0, bc0, 'elu'),
      (U1, S1, Wc1, bc1, 'elu'),
      (U0, S0, Wc2, bc2, None),
  ]
  out_col = None
  for (U, S, Wc, bc, act), (C, mb_up, mb_cv) in zip(specs, _LEVEL_TUNE):
    M = U.shape[0]
    F = hcol.shape[1] // B
    sl = S.shape[-1]
    OC = Wc.shape[1]

    y = _mm(U, hcol, mb=mb_up)

    gidx = S[0].reshape(-1)
    G = gidx.shape[0]
    rpw = _cdiv(_cdiv(G, C), _NW)
    Gp = _NW * rpw * C
    if Gp != G:
      gidx = jnp.concatenate([gidx, jnp.zeros((Gp - G,), jnp.int32)])
    gout = _sc_gather(y, gidx, B * F, C, rpw)
    R = Gp // sl
    gmat = gout.reshape(R, sl * B * F)

    W3 = Wc.reshape(sl, F, OC)
    eyeB = jnp.eye(B, dtype=jnp.float32)
    W2 = jnp.einsum('sco,bd->sbcdo', W3, eyeB).reshape(sl * B * F, B * OC)
    b2 = jnp.tile(bc, B)

    rows = jnp.arange(R, dtype=jnp.int32)
    mcol = jnp.where(rows == M - 1, 0.0, 1.0).astype(jnp.float32).reshape(R, 1)

    out_col = _mm(gmat, W2, bias=b2, act=act, mask=mcol, mb=mb_cv)
    hcol = out_col[:M]
    M_last, OC_last = M, OC
  return (out_col[:M_last]
          .reshape(M_last, B, OC_last)
          .transpose(1, 0, 2))


# speedup vs baseline: 1.1230x; 1.1230x over previous
import functools
import math

import jax
import jax.numpy as jnp
from jax import lax
from jax.experimental import pallas as pl
from jax.experimental.pallas import tpu as pltpu
from jax.experimental.pallas import tpu_sc as plsc

_NC, _NS = 2, 16
_NW = _NC * _NS


def _cdiv(a, b):
  return (a + b - 1) // b


def _mm(x, w, *, bias=None, act=None, mask=None, mb=None):
  M, K = x.shape
  N = w.shape[1]
  if mb is None or mb >= M:
    mb = M
  grid = (_cdiv(M, mb),)
  in_specs = [
      pl.BlockSpec((mb, K), lambda i: (i, 0)),
      pl.BlockSpec((K, N), lambda i: (0, 0)),
  ]
  args = [x, w]
  if bias is not None:
    in_specs.append(pl.BlockSpec((1, N), lambda i: (0, 0)))
    args.append(bias.reshape(1, N))
  if mask is not None:
    in_specs.append(pl.BlockSpec((mb, 1), lambda i: (i, 0)))
    args.append(mask)

  def kern(*refs):
    x_ref, w_ref = refs[0], refs[1]
    o_ref = refs[-1]
    acc = jnp.dot(x_ref[...], w_ref[...], preferred_element_type=jnp.float32)
    p = 2
    if bias is not None:
      acc = acc + refs[p][...]
      p += 1
    if act == 'elu':
      acc = jnp.where(acc > 0, acc, jnp.exp(jnp.minimum(acc, 0.0)) - 1.0)
    if mask is not None:
      acc = acc * refs[p][...]
      p += 1
    o_ref[...] = acc

  return pl.pallas_call(
      kern,
      grid=grid,
      in_specs=in_specs,
      out_specs=pl.BlockSpec((mb, N), lambda i: (i, 0)),
      out_shape=jax.ShapeDtypeStruct((M, N), jnp.float32),
  )(*args)


def _sc_gather(table, gidx, F, C, rpw):
  Gp = gidx.shape[0]
  assert Gp == _NW * rpw * C
  mesh = plsc.VectorSubcoreMesh(
      core_axis_name="c", subcore_axis_name="s",
      num_cores=_NC, num_subcores=_NS)

  @functools.partial(
      pl.kernel,
      out_type=jax.ShapeDtypeStruct((Gp, F), jnp.float32),
      mesh=mesh,
      scratch_types=[
          pltpu.VMEM((C,), jnp.int32),
          pltpu.VMEM((C,), jnp.int32),
          pltpu.VMEM((C, F), jnp.float32),
          pltpu.VMEM((C, F), jnp.float32),
      ] + [pltpu.SemaphoreType.DMA] * 6,
  )
  def gk(table_hbm, gidx_hbm, out_hbm, ib0, ib1, buf0, buf1,
         si0, si1, sg0, sg1, so0, so1):
    wid = lax.axis_index("s") * _NC + lax.axis_index("c")
    ibufs = (ib0, ib1)
    bufs = (buf0, buf1)
    isems = (si0, si1)
    gsems = (sg0, sg1)
    osems = (so0, so1)

    def chunk(j):
      return j * _NW + wid

    def fire_idx(j):
      return pltpu.async_copy(
          gidx_hbm.at[pl.ds(chunk(j) * C, C)], ibufs[j % 2], isems[j % 2])

    def fire_in(j):
      return pltpu.async_copy(
          table_hbm.at[ibufs[j % 2]], bufs[j % 2], gsems[j % 2])

    def fire_out(j):
      return pltpu.async_copy(
          bufs[j % 2], out_hbm.at[pl.ds(chunk(j) * C, C)], osems[j % 2])

    idxs = [None] * rpw
    ins = [None] * rpw
    outs = [None] * rpw
    idxs[0] = fire_idx(0)
    for j in range(rpw):
      if j + 1 < rpw:
        idxs[j + 1] = fire_idx(j + 1)
      idxs[j].wait()
      if j - 2 >= 0:
        outs[j - 2].wait()
      ins[j] = fire_in(j)
      ins[j].wait()
      outs[j] = fire_out(j)
    if rpw >= 2:
      outs[rpw - 2].wait()
    outs[rpw - 1].wait()

  return gk(table, gidx)


_LEVEL_TUNE = [
    (120, None, None),
    (240, None, 640),
    (400, 1112, 1280),
]


def kernel(x, W_fc, b_fc, U0, U1, U2, S0, S1, S2, Wc0, bc0, Wc1, bc1, Wc2, bc2):
  B = x.shape[0]
  h = _mm(x, W_fc, bias=b_fc)
  M_in = U2.shape[1]
  F_in = h.shape[1] // M_in
  hcol = h.reshape(B, M_in, F_in).transpose(1, 0, 2).reshape(M_in, B * F_in)

  specs = [
      (U2, S2, Wc0, bc0, 'elu'),
      (U1, S1, Wc1, bc1, 'elu'),
      (U0, S0, Wc2, bc2, None),
  ]
  out_col = None
  for (U, S, Wc, bc, act), (C, mb_up, mb_cv) in zip(specs, _LEVEL_TUNE):
    M = U.shape[0]
    F = hcol.shape[1] // B
    sl = S.shape[-1]
    OC = Wc.shape[1]

    y = _mm(U, hcol, mb=mb_up)

    gidx = S[0].reshape(-1)
    G = gidx.shape[0]
    rpw = _cdiv(_cdiv(G, C), _NW)
    Gp = _NW * rpw * C
    if Gp != G:
      gidx = jnp.concatenate([gidx, jnp.zeros((Gp - G,), jnp.int32)])
    gout = _sc_gather(y, gidx, B * F, C, rpw)
    R = Gp // sl
    gmat = gout.reshape(R, sl * B * F)

    W3 = Wc.reshape(sl, F, OC)
    eyeB = jnp.eye(B, dtype=jnp.float32)
    W2 = jnp.einsum('sco,bd->sbcdo', W3, eyeB).reshape(sl * B * F, B * OC)
    b2 = jnp.tile(bc, B)

    rows = jnp.arange(R, dtype=jnp.int32)
    mcol = jnp.where(rows == M - 1, 0.0, 1.0).astype(jnp.float32).reshape(R, 1)

    out_col = _mm(gmat, W2, bias=b2, act=act, mask=mcol, mb=mb_cv)
    hcol = out_col[:M]
    M_last, OC_last = M, OC
  return (out_col[:M_last]
          .reshape(M_last, B, OC_last)
          .transpose(1, 0, 2))


# speedup vs baseline: 1.3342x; 1.1881x over previous
import functools
import math

import jax
import jax.numpy as jnp
from jax import lax
from jax.experimental import pallas as pl
from jax.experimental.pallas import tpu as pltpu
from jax.experimental.pallas import tpu_sc as plsc

_NC, _NS = 2, 16
_NW = _NC * _NS


def _cdiv(a, b):
  return (a + b - 1) // b


def _mm(x, w, *, bias=None, act=None, mask=None, mb=None):
  M, K = x.shape
  N = w.shape[1]
  if mb is None or mb >= M:
    mb = M
  grid = (_cdiv(M, mb),)
  in_specs = [
      pl.BlockSpec((mb, K), lambda i: (i, 0)),
      pl.BlockSpec((K, N), lambda i: (0, 0)),
  ]
  args = [x, w]
  if bias is not None:
    in_specs.append(pl.BlockSpec((1, N), lambda i: (0, 0)))
    args.append(bias.reshape(1, N))
  if mask is not None:
    in_specs.append(pl.BlockSpec((mb, 1), lambda i: (i, 0)))
    args.append(mask)

  def kern(*refs):
    x_ref, w_ref = refs[0], refs[1]
    o_ref = refs[-1]
    acc = jnp.dot(x_ref[...], w_ref[...], preferred_element_type=jnp.float32)
    p = 2
    if bias is not None:
      acc = acc + refs[p][...]
      p += 1
    if act == 'elu':
      acc = jnp.where(acc > 0, acc, jnp.exp(jnp.minimum(acc, 0.0)) - 1.0)
    if mask is not None:
      acc = acc * refs[p][...]
      p += 1
    o_ref[...] = acc

  return pl.pallas_call(
      kern,
      grid=grid,
      in_specs=in_specs,
      out_specs=pl.BlockSpec((mb, N), lambda i: (i, 0)),
      out_shape=jax.ShapeDtypeStruct((M, N), jnp.float32),
  )(*args)


def _conv_mm(g2d, w3, bias, act, mask, mb, sl):
  Gp, BF = g2d.shape
  R = Gp // sl
  OC2 = w3.shape[2]
  grid = (_cdiv(R, mb),)

  def kern(x_ref, w_ref, b_ref, m_ref, o_ref):
    x3 = x_ref[...].reshape(mb, sl, BF)
    acc = jnp.dot(x3[:, 0, :], w_ref[0], preferred_element_type=jnp.float32)
    for s in range(1, sl):
      acc += jnp.dot(x3[:, s, :], w_ref[s], preferred_element_type=jnp.float32)
    acc = acc + b_ref[...]
    if act == 'elu':
      acc = jnp.where(acc > 0, acc, jnp.exp(jnp.minimum(acc, 0.0)) - 1.0)
    o_ref[...] = acc * m_ref[...]

  return pl.pallas_call(
      kern,
      grid=grid,
      in_specs=[
          pl.BlockSpec((mb * sl, BF), lambda i: (i, 0)),
          pl.BlockSpec((sl, BF, OC2), lambda i: (0, 0, 0)),
          pl.BlockSpec((1, OC2), lambda i: (0, 0)),
          pl.BlockSpec((mb, 1), lambda i: (i, 0)),
      ],
      out_specs=pl.BlockSpec((mb, OC2), lambda i: (i, 0)),
      out_shape=jax.ShapeDtypeStruct((R, OC2), jnp.float32),
  )(g2d, w3, bias.reshape(1, OC2), mask)


def _sc_gather(table, gidx, F, C, rpw):
  Gp = gidx.shape[0]
  assert Gp == _NW * rpw * C
  mesh = plsc.VectorSubcoreMesh(
      core_axis_name="c", subcore_axis_name="s",
      num_cores=_NC, num_subcores=_NS)

  @functools.partial(
      pl.kernel,
      out_type=jax.ShapeDtypeStruct((Gp, F), jnp.float32),
      mesh=mesh,
      scratch_types=[
          pltpu.VMEM((C,), jnp.int32),
          pltpu.VMEM((C,), jnp.int32),
          pltpu.VMEM((C, F), jnp.float32),
      ] + [pltpu.SemaphoreType.DMA] * 4,
  )
  def gk(table_hbm, gidx_hbm, out_hbm, ib0, ib1, buf,
         si0, si1, sg, so):
    wid = lax.axis_index("s") * _NC + lax.axis_index("c")
    ibufs = (ib0, ib1)
    isems = (si0, si1)

    def chunk(j):
      return j * _NW + wid

    def fire_idx(j):
      return pltpu.async_copy(
          gidx_hbm.at[pl.ds(chunk(j) * C, C)], ibufs[j % 2], isems[j % 2])

    idxs = [None] * rpw
    outs = [None] * rpw
    idxs[0] = fire_idx(0)
    for j in range(rpw):
      if j + 1 < rpw:
        idxs[j + 1] = fire_idx(j + 1)
      idxs[j].wait()
      if j - 1 >= 0:
        outs[j - 1].wait()
      pltpu.async_copy(table_hbm.at[ibufs[j % 2]], buf, sg).wait()
      outs[j] = pltpu.async_copy(
          buf, out_hbm.at[pl.ds(chunk(j) * C, C)], so)
    outs[rpw - 1].wait()

  return gk(table, gidx)


_LEVEL_TUNE = [
    (120, None, 640),
    (240, None, 640),
    (640, 1112, 1280),
]


def kernel(x, W_fc, b_fc, U0, U1, U2, S0, S1, S2, Wc0, bc0, Wc1, bc1, Wc2, bc2):
  B = x.shape[0]
  h = _mm(x, W_fc, bias=b_fc)
  M_in = U2.shape[1]
  F_in = h.shape[1] // M_in
  hcol = h.reshape(B, M_in, F_in).transpose(1, 0, 2).reshape(M_in, B * F_in)

  specs = [
      (U2, S2, Wc0, bc0, 'elu'),
      (U1, S1, Wc1, bc1, 'elu'),
      (U0, S0, Wc2, bc2, None),
  ]
  out_col = None
  for (U, S, Wc, bc, act), (C, mb_up, mb_cv) in zip(specs, _LEVEL_TUNE):
    M = U.shape[0]
    F = hcol.shape[1] // B
    sl = S.shape[-1]
    OC = Wc.shape[1]

    y = _mm(U, hcol, mb=mb_up)

    gidx = S[0].reshape(-1)
    G = gidx.shape[0]
    rpw = _cdiv(_cdiv(G, C), _NW)
    Gp = _NW * rpw * C
    if Gp != G:
      gidx = jnp.concatenate([gidx, jnp.zeros((Gp - G,), jnp.int32)])
    gout = _sc_gather(y, gidx, B * F, C, rpw)
    R = Gp // sl

    W3 = Wc.reshape(sl, F, OC)
    eyeB = jnp.eye(B, dtype=jnp.float32)
    W2 = jnp.einsum('sco,bd->sbcdo', W3, eyeB).reshape(sl, B * F, B * OC)
    b2 = jnp.tile(bc, B)

    rows = jnp.arange(R, dtype=jnp.int32)
    mcol = jnp.where(rows == M - 1, 0.0, 1.0).astype(jnp.float32).reshape(R, 1)

    out_col = _conv_mm(gout, W2, b2, act, mcol, mb_cv, sl)
    hcol = out_col[:M]
    M_last, OC_last = M, OC
  return (out_col[:M_last]
          .reshape(M_last, B, OC_last)
          .transpose(1, 0, 2))


# speedup vs baseline: 1.7493x; 1.3112x over previous
import functools
import math

import jax
import jax.numpy as jnp
from jax import lax
from jax.experimental import pallas as pl
from jax.experimental.pallas import tpu as pltpu
from jax.experimental.pallas import tpu_sc as plsc

_NC, _NS = 2, 16
_NW = _NC * _NS


def _cdiv(a, b):
  return (a + b - 1) // b


def _mm(x, w, *, bias=None, act=None, mask=None, mb=None):
  M, K = x.shape
  N = w.shape[1]
  if mb is None or mb >= M:
    mb = M
  grid = (_cdiv(M, mb),)
  in_specs = [
      pl.BlockSpec((mb, K), lambda i: (i, 0)),
      pl.BlockSpec((K, N), lambda i: (0, 0)),
  ]
  args = [x, w]
  if bias is not None:
    in_specs.append(pl.BlockSpec((1, N), lambda i: (0, 0)))
    args.append(bias.reshape(1, N))
  if mask is not None:
    in_specs.append(pl.BlockSpec((mb, 1), lambda i: (i, 0)))
    args.append(mask)

  def kern(*refs):
    x_ref, w_ref = refs[0], refs[1]
    o_ref = refs[-1]
    acc = jnp.dot(x_ref[...], w_ref[...], preferred_element_type=jnp.float32)
    p = 2
    if bias is not None:
      acc = acc + refs[p][...]
      p += 1
    if act == 'elu':
      acc = jnp.where(acc > 0, acc, jnp.exp(jnp.minimum(acc, 0.0)) - 1.0)
    if mask is not None:
      acc = acc * refs[p][...]
      p += 1
    o_ref[...] = acc

  return pl.pallas_call(
      kern,
      grid=grid,
      in_specs=in_specs,
      out_specs=pl.BlockSpec((mb, N), lambda i: (i, 0)),
      out_shape=jax.ShapeDtypeStruct((M, N), jnp.float32),
  )(*args)


def _up_mm(Ut, hcol, nb):
  K, M = Ut.shape
  BF = hcol.shape[1]
  if nb is None or nb >= M:
    nb = M
  grid = (_cdiv(M, nb),)

  def kern(u_ref, h_ref, o_ref):
    o_ref[...] = lax.dot_general(
        u_ref[...], h_ref[...], (((0,), (0,)), ((), ())),
        preferred_element_type=jnp.float32)

  return pl.pallas_call(
      kern,
      grid=grid,
      in_specs=[
          pl.BlockSpec((K, nb), lambda i: (0, i)),
          pl.BlockSpec((K, BF), lambda i: (0, 0)),
      ],
      out_specs=pl.BlockSpec((nb, BF), lambda i: (i, 0)),
      out_shape=jax.ShapeDtypeStruct((M, BF), jnp.float32),
  )(Ut, hcol)


def _conv_mm(g2d, w3, bias, act, mask, mb, sl):
  Gp, BF = g2d.shape
  R = Gp // sl
  OC2 = w3.shape[2]
  grid = (_cdiv(R, mb),)

  def kern(x_ref, w_ref, b_ref, m_ref, o_ref):
    x3 = x_ref[...].reshape(mb, sl, BF)
    acc = jnp.dot(x3[:, 0, :], w_ref[0], preferred_element_type=jnp.float32)
    for s in range(1, sl):
      acc += jnp.dot(x3[:, s, :], w_ref[s], preferred_element_type=jnp.float32)
    acc = acc + b_ref[...]
    if act == 'elu':
      acc = jnp.where(acc > 0, acc, jnp.exp(jnp.minimum(acc, 0.0)) - 1.0)
    o_ref[...] = acc * m_ref[...]

  return pl.pallas_call(
      kern,
      grid=grid,
      in_specs=[
          pl.BlockSpec((mb * sl, BF), lambda i: (i, 0)),
          pl.BlockSpec((sl, BF, OC2), lambda i: (0, 0, 0)),
          pl.BlockSpec((1, OC2), lambda i: (0, 0)),
          pl.BlockSpec((mb, 1), lambda i: (i, 0)),
      ],
      out_specs=pl.BlockSpec((mb, OC2), lambda i: (i, 0)),
      out_shape=jax.ShapeDtypeStruct((R, OC2), jnp.float32),
  )(g2d, w3, bias.reshape(1, OC2), mask)


def _sc_gather(table, gidx, F, C, nch):
  Gp = gidx.shape[0]
  assert Gp == nch * C
  rounds = _cdiv(nch, _NW)
  mesh = plsc.VectorSubcoreMesh(
      core_axis_name="c", subcore_axis_name="s",
      num_cores=_NC, num_subcores=_NS)

  @functools.partial(
      pl.kernel,
      out_type=jax.ShapeDtypeStruct((Gp, F), jnp.float32),
      mesh=mesh,
      scratch_types=[
          pltpu.VMEM((C,), jnp.int32),
          pltpu.VMEM((C, F), jnp.float32),
          pltpu.SemaphoreType.DMA,
      ],
  )
  def gk(table_hbm, gidx_hbm, out_hbm, idx_v, rows_v, sem):
    wid = lax.axis_index("s") * _NC + lax.axis_index("c")

    def body(j, carry):
      ch = j * _NW + wid

      @pl.when(ch < nch)
      def _():
        base = ch * C
        pltpu.sync_copy(gidx_hbm.at[pl.ds(base, C)], idx_v)
        pltpu.async_copy(table_hbm.at[idx_v], rows_v, sem).wait()
        pltpu.sync_copy(rows_v, out_hbm.at[pl.ds(base, C)])

      return carry

    lax.fori_loop(0, rounds, body, 0)

  return gk(table, gidx)


_LEVEL_TUNE = [
    (128, None, 640),
    (256, None, 640),
    (512, 1024, 1264),
]


def kernel(x, W_fc, b_fc, U0, U1, U2, S0, S1, S2, Wc0, bc0, Wc1, bc1, Wc2, bc2):
  B = x.shape[0]
  h = _mm(x, W_fc, bias=b_fc)
  M_in = U2.shape[1]
  F_in = h.shape[1] // M_in
  hcol = h.reshape(B, M_in, F_in).transpose(1, 0, 2).reshape(M_in, B * F_in)

  specs = [
      (U2, S2, Wc0, bc0, 'elu'),
      (U1, S1, Wc1, bc1, 'elu'),
      (U0, S0, Wc2, bc2, None),
  ]
  out_col = None
  for (U, S, Wc, bc, act), (C, nb_up, mb_cv) in zip(specs, _LEVEL_TUNE):
    M = U.shape[0]
    F = hcol.shape[1] // B
    sl = S.shape[-1]
    OC = Wc.shape[1]

    y = _up_mm(U.T, hcol, nb_up)

    gidx = S[0].reshape(-1)
    G = gidx.shape[0]
    step = math.lcm(C, sl)
    Gp = _cdiv(G, step) * step
    nch = Gp // C
    if Gp != G:
      gidx = jnp.concatenate([gidx, jnp.zeros((Gp - G,), jnp.int32)])
    gout = _sc_gather(y, gidx, B * F, C, nch)
    R = Gp // sl

    W3 = Wc.reshape(sl, F, OC)
    eyeB = jnp.eye(B, dtype=jnp.float32)
    W2 = jnp.einsum('sco,bd->sbcdo', W3, eyeB).reshape(sl, B * F, B * OC)
    b2 = jnp.tile(bc, B)

    rows = jnp.arange(R, dtype=jnp.int32)
    mcol = jnp.where(rows == M - 1, 0.0, 1.0).astype(jnp.float32).reshape(R, 1)

    out_col = _conv_mm(gout, W2, b2, act, mcol, mb_cv, sl)
    hcol = out_col[:M]
    M_last, OC_last = M, OC
  return (out_col[:M_last]
          .reshape(M_last, B, OC_last)
          .transpose(1, 0, 2))
